# baseline (speedup 1.0000x reference)
import functools

import jax
import jax.numpy as jnp
from jax import lax
from jax.experimental import pallas as pl
from jax.experimental.pallas import tpu as pltpu
from jax.experimental.pallas import tpu_sc as plsc

_B = 16384
_F = 26
_V = 100000
_D = 16

_L = 16
_NC = 2
_NS = 16
_NW = _NC * _NS
_BPW = _B // _NW
_CB = 128
_NCHUNK = _BPW // _CB
_NIDX = _CB * _F
_NG = _CB // _L


@functools.partial(
    pl.kernel,
    out_type=jax.ShapeDtypeStruct((_B,), jnp.float32),
    mesh=plsc.VectorSubcoreMesh(core_axis_name="c", subcore_axis_name="s"),
    compiler_params=pltpu.CompilerParams(
        needs_layout_passes=False, use_tc_tiling_on_sc=False),
    scratch_types=[
        pltpu.VMEM((_F, _CB), jnp.int32),
        pltpu.VMEM((_NIDX,), jnp.int32),
        pltpu.VMEM((_D, _NIDX), jnp.float32),
        pltpu.VMEM((_NIDX,), jnp.float32),
        pltpu.VMEM((_BPW,), jnp.float32),
        pltpu.SemaphoreType.DMA,
        pltpu.SemaphoreType.DMA,
    ],
)
def _fm_kernel(idx_hbm, lin_hbm, emb_hbm,
               out_hbm, idx2_v, idxf_v, rowsT_v, lin_v, out_v, sem_e, sem_l):
    cid = lax.axis_index("c")
    sid = lax.axis_index("s")
    wid = sid * _NC + cid
    wbase = wid * _BPW

    def chunk_body(c, carry):
        bbase = wbase + c * _CB
        pltpu.sync_copy(idx_hbm.at[:, pl.ds(bbase, _CB)], idx2_v)

        def flat_body(j, carry2):
            f = j // _NG
            g = j - f * _NG
            col = g * _L
            v = idx2_v[f, pl.ds(col, _L)] + f * _V
            idxf_v[pl.ds(pl.multiple_of(f * _CB + col, _L), _L)] = v
            return carry2

        lax.fori_loop(0, _F * _NG, flat_body, 0)

        copies = []
        for d in range(_D):
            copies.append(pltpu.make_async_copy(
                emb_hbm.at[pl.ds(d * _F * _V, _F * _V)].at[idxf_v],
                rowsT_v.at[d], sem_e))
        copies.append(pltpu.make_async_copy(
            lin_hbm.at[idxf_v], lin_v, sem_l))
        for cp in copies:
            cp.start()
        for cp in copies:
            cp.wait()

        def comp_body(g, carry2):
            col = g * _L
            ss = jnp.zeros((_L,), jnp.float32)
            s2 = jnp.zeros((_L,), jnp.float32)
            for d in range(_D):
                s = jnp.zeros((_L,), jnp.float32)
                for f in range(_F):
                    e = rowsT_v[d, pl.ds(f * _CB + col, _L)]
                    s = s + e
                    ss = ss + e * e
                s2 = s2 + s * s
            lin = jnp.zeros((_L,), jnp.float32)
            for f in range(_F):
                lin = lin + lin_v[pl.ds(f * _CB + col, _L)]
            out_v[pl.ds(c * _CB + col, _L)] = (
                jnp.float32(0.5) * (s2 - ss) + lin)
            return carry2

        lax.fori_loop(0, _NG, comp_body, 0)
        return carry

    lax.fori_loop(0, _NCHUNK, chunk_body, 0)
    pltpu.sync_copy(out_v, out_hbm.at[pl.ds(wbase, _BPW)])


def kernel(indices, linear_table, embed_table, bias):
    idx_t = indices.astype(jnp.int32).T
    lin_col = linear_table.T.reshape(_F * _V)
    emb_dmaj = embed_table.T.reshape(_D * _F * _V)
    out = _fm_kernel(idx_t, lin_col, emb_dmaj)
    return bias + out


# speedup vs baseline: 7.0391x; 7.0391x over previous
import functools

import jax
import jax.numpy as jnp
from jax import lax
from jax.experimental import pallas as pl
from jax.experimental.pallas import tpu as pltpu
from jax.experimental.pallas import tpu_sc as plsc

_B = 16384
_F = 26
_V = 100000
_D = 16

_L = 16
_NC = 2
_NS = 16
_NW = _NC * _NS
_BPW = _B // _NW
_CB = 128
_NCHUNK = _BPW // _CB
_NIDX = _CB * _F
_NG = _CB // _L


@functools.partial(
    pl.kernel,
    out_type=jax.ShapeDtypeStruct((_B,), jnp.float32),
    mesh=plsc.VectorSubcoreMesh(core_axis_name="c", subcore_axis_name="s"),
    compiler_params=pltpu.CompilerParams(
        needs_layout_passes=False, use_tc_tiling_on_sc=False),
    scratch_types=[
        pltpu.VMEM((_F, _CB), jnp.int32),
        pltpu.VMEM((_NIDX,), jnp.int32),
        pltpu.VMEM((_D, _NIDX), jnp.float32),
        pltpu.VMEM((_NIDX,), jnp.float32),
        pltpu.VMEM((_BPW,), jnp.float32),
        pltpu.SemaphoreType.DMA,
        pltpu.SemaphoreType.DMA,
    ],
)
def _fm_kernel(idx_hbm, lin_hbm,
               e0, e1, e2, e3, e4, e5, e6, e7,
               e8, e9, e10, e11, e12, e13, e14, e15,
               out_hbm, idx2_v, idxf_v, rowsT_v, lin_v, out_v, sem_e, sem_l):
    embs = (e0, e1, e2, e3, e4, e5, e6, e7,
            e8, e9, e10, e11, e12, e13, e14, e15)
    cid = lax.axis_index("c")
    sid = lax.axis_index("s")
    wid = sid * _NC + cid
    wbase = wid * _BPW

    def chunk_body(c, carry):
        bbase = wbase + c * _CB
        pltpu.sync_copy(idx_hbm.at[:, pl.ds(bbase, _CB)], idx2_v)

        def flat_body(j, carry2):
            f = j // _NG
            g = j - f * _NG
            col = g * _L
            v = idx2_v[f, pl.ds(col, _L)] + f * _V
            idxf_v[pl.ds(pl.multiple_of(f * _CB + col, _L), _L)] = v
            return carry2

        lax.fori_loop(0, _F * _NG, flat_body, 0)

        copies = []
        for d in range(_D):
            copies.append(pltpu.make_async_copy(
                embs[d].at[idxf_v], rowsT_v.at[d], sem_e))
        copies.append(pltpu.make_async_copy(
            lin_hbm.at[idxf_v], lin_v, sem_l))
        for cp in copies:
            cp.start()
        for cp in copies:
            cp.wait()

        def comp_body(g, carry2):
            col = g * _L
            ss = jnp.zeros((_L,), jnp.float32)
            s2 = jnp.zeros((_L,), jnp.float32)
            for d in range(_D):
                s = jnp.zeros((_L,), jnp.float32)
                for f in range(_F):
                    e = rowsT_v[d, pl.ds(f * _CB + col, _L)]
                    s = s + e
                    ss = ss + e * e
                s2 = s2 + s * s
            lin = jnp.zeros((_L,), jnp.float32)
            for f in range(_F):
                lin = lin + lin_v[pl.ds(f * _CB + col, _L)]
            out_v[pl.ds(c * _CB + col, _L)] = (
                jnp.float32(0.5) * (s2 - ss) + lin)
            return carry2

        lax.fori_loop(0, _NG, comp_body, 0)
        return carry

    lax.fori_loop(0, _NCHUNK, chunk_body, 0)
    pltpu.sync_copy(out_v, out_hbm.at[pl.ds(wbase, _BPW)])


def kernel(indices, linear_table, embed_table, bias):
    idx_t = indices.astype(jnp.int32).T
    lin_col = linear_table.T.reshape(_F * _V)
    emb_cols = _split_columns(embed_table.T)
    out = _fm_kernel(idx_t, lin_col, *emb_cols)
    return bias + out


_SPLIT_BK = 131072


def _split_body(in_ref, *out_refs):
    for d in range(_D):
        out_refs[d][...] = in_ref[d, :]


_split_columns = pl.pallas_call(
    _split_body,
    grid=((_F * _V + _SPLIT_BK - 1) // _SPLIT_BK,),
    in_specs=[pl.BlockSpec((_D, _SPLIT_BK), lambda i: (0, i))],
    out_specs=[pl.BlockSpec((_SPLIT_BK,), lambda i: (i,))] * _D,
    out_shape=[jax.ShapeDtypeStruct((_F * _V,), jnp.float32)] * _D,
)


# speedup vs baseline: 8.0566x; 1.1445x over previous
import functools

import jax
import jax.numpy as jnp
from jax import lax
from jax.experimental import pallas as pl
from jax.experimental.pallas import tpu as pltpu
from jax.experimental.pallas import tpu_sc as plsc

_B = 16384
_F = 26
_V = 100000
_D = 16

_L = 16
_NC = 2
_NS = 16
_NW = _NC * _NS
_BPW = _B // _NW
_CB = 128
_NCHUNK = _BPW // _CB
_NIDX = _CB * _F
_NG = _CB // _L
_DG = 4

_sc_mesh = plsc.VectorSubcoreMesh(core_axis_name="c", subcore_axis_name="s")
_sc_params = pltpu.CompilerParams(
    needs_layout_passes=False, use_tc_tiling_on_sc=False)


def _stage_indices(idx_hbm, idx2_v, idxf_v, c, wbase):
    bbase = wbase + c * _CB
    pltpu.sync_copy(idx_hbm.at[:, pl.ds(bbase, _CB)], idx2_v)

    def flat_body(j, carry):
        f = j // _NG
        g = j - f * _NG
        col = g * _L
        v = idx2_v[f, pl.ds(col, _L)] + f * _V
        idxf_v[pl.ds(pl.multiple_of(f * _CB + col, _L), _L)] = v
        return carry

    lax.fori_loop(0, _F * _NG, flat_body, 0)


@functools.partial(
    pl.kernel,
    out_type=jax.ShapeDtypeStruct((_B,), jnp.float32),
    mesh=_sc_mesh,
    compiler_params=_sc_params,
    scratch_types=[
        pltpu.VMEM((_F, _CB), jnp.int32),
        pltpu.VMEM((_NIDX,), jnp.int32),
        pltpu.VMEM((_DG, _NIDX), jnp.float32),
        pltpu.VMEM((_BPW,), jnp.float32),
        pltpu.SemaphoreType.DMA,
    ],
)
def _fm_partial(idx_hbm, e0, e1, e2, e3,
                out_hbm, idx2_v, idxf_v, rowsT_v, out_v, sem):
    embs = (e0, e1, e2, e3)
    wid = lax.axis_index("s") * _NC + lax.axis_index("c")
    wbase = wid * _BPW

    def chunk_body(c, carry):
        _stage_indices(idx_hbm, idx2_v, idxf_v, c, wbase)
        copies = [pltpu.make_async_copy(embs[d].at[idxf_v],
                                        rowsT_v.at[d], sem)
                  for d in range(_DG)]
        for cp in copies:
            cp.start()
        for cp in copies:
            cp.wait()

        def comp_body(g, carry2):
            col = g * _L
            acc = jnp.zeros((_L,), jnp.float32)
            for d in range(_DG):
                s = jnp.zeros((_L,), jnp.float32)
                for f in range(_F):
                    e = rowsT_v[d, pl.ds(f * _CB + col, _L)]
                    s = s + e
                    acc = acc - e * e
                acc = acc + s * s
            out_v[pl.ds(c * _CB + col, _L)] = acc
            return carry2

        lax.fori_loop(0, _NG, comp_body, 0)
        return carry

    lax.fori_loop(0, _NCHUNK, chunk_body, 0)
    pltpu.sync_copy(out_v, out_hbm.at[pl.ds(wbase, _BPW)])


@functools.partial(
    pl.kernel,
    out_type=jax.ShapeDtypeStruct((_B,), jnp.float32),
    mesh=_sc_mesh,
    compiler_params=_sc_params,
    scratch_types=[
        pltpu.VMEM((_F, _CB), jnp.int32),
        pltpu.VMEM((_NIDX,), jnp.int32),
        pltpu.VMEM((_NIDX,), jnp.float32),
        pltpu.VMEM((_BPW,), jnp.float32),
        pltpu.SemaphoreType.DMA,
    ],
)
def _fm_linear(idx_hbm, lin_hbm, out_hbm, idx2_v, idxf_v, lin_v, out_v, sem):
    wid = lax.axis_index("s") * _NC + lax.axis_index("c")
    wbase = wid * _BPW

    def chunk_body(c, carry):
        _stage_indices(idx_hbm, idx2_v, idxf_v, c, wbase)
        pltpu.async_copy(lin_hbm.at[idxf_v], lin_v, sem).wait()

        def comp_body(g, carry2):
            col = g * _L
            acc = jnp.zeros((_L,), jnp.float32)
            for f in range(_F):
                acc = acc + lin_v[pl.ds(f * _CB + col, _L)]
            out_v[pl.ds(c * _CB + col, _L)] = acc
            return carry2

        lax.fori_loop(0, _NG, comp_body, 0)
        return carry

    lax.fori_loop(0, _NCHUNK, chunk_body, 0)
    pltpu.sync_copy(out_v, out_hbm.at[pl.ds(wbase, _BPW)])


_SPLIT_BK = 131072


_SG = 8


def _split_body(in_ref, *out_refs):
    for d in range(_SG):
        out_refs[d][...] = in_ref[d, :]


def _make_splitter(dgroup):
    return pl.pallas_call(
        _split_body,
        grid=((_F * _V + _SPLIT_BK - 1) // _SPLIT_BK,),
        in_specs=[pl.BlockSpec((_SG, _SPLIT_BK),
                               lambda i, dg=dgroup: (dg, i))],
        out_specs=[pl.BlockSpec((_SPLIT_BK,), lambda i: (i,))] * _SG,
        out_shape=[jax.ShapeDtypeStruct((_F * _V,), jnp.float32)] * _SG,
    )


_splitters = [_make_splitter(k) for k in range(_D // _SG)]


def kernel(indices, linear_table, embed_table, bias):
    idx_t = indices.astype(jnp.int32).T
    lin_col = linear_table.T.reshape(_F * _V)
    emb_t = embed_table.T
    partials = []
    for k in range(_D // _SG):
        cols = _splitters[k](emb_t)
        partials.append(_fm_partial(idx_t, *cols[:_DG]))
        partials.append(_fm_partial(idx_t, *cols[_DG:]))
    lin_out = _fm_linear(idx_t, lin_col)
    second = partials[0] + partials[1] + partials[2] + partials[3]
    return bias + jnp.float32(0.5) * second + lin_out
